# Initial kernel scaffold; baseline (speedup 1.0000x reference)
import jax
import jax.numpy as jnp
from jax.experimental import pallas as pl


def kernel(x, edge_index, W1, b1, W2, b2, temp):
    raise NotImplementedError("write your pallas kernel here")


# speedup vs baseline: 11.2723x; 11.2723x over previous
import functools

import jax
import jax.numpy as jnp
from jax import lax
from jax.experimental import pallas as pl
from jax.experimental.pallas import tpu as pltpu
from jax.experimental.pallas import tpu_sc as plsc

N_NODES = 10000
N_PAD = 10240
N_EDGES = 320000
D = 128
K_STEPS = 10

NC = 2
NS = 16
NT = NC * NS
EDGES_PER_TILE = N_EDGES // NT
CHUNK = 80
NCHUNK = EDGES_PER_TILE // CHUNK
ROWS_PER_TILE = N_PAD // NS
WR_CHUNK = 128
NWR = ROWS_PER_TILE // WR_CHUNK

_mesh = plsc.VectorSubcoreMesh(
    core_axis_name="c", subcore_axis_name="s", num_cores=NC, num_subcores=NS
)


@functools.partial(
    pl.kernel,
    out_type=jax.ShapeDtypeStruct((NC, N_PAD, D), jnp.float32),
    mesh=_mesh,
    scratch_types=[
        pltpu.VMEM((NCHUNK, CHUNK), jnp.int32),
        pltpu.VMEM((CHUNK, D), jnp.float32),
        pltpu.VMEM_SHARED((N_PAD, D), jnp.float32),
    ],
)
def _deg_kernel(cols_hbm, out_hbm, colbuf, gbuf, acc):
    cid = lax.axis_index("c")
    sid = lax.axis_index("s")
    wid = cid * NS + sid

    @pl.loop(0, CHUNK)
    def _(i):
        for j in range(D // 16):
            gbuf[i, pl.ds(j * 16, 16)] = jnp.zeros((16,), jnp.float32)

    base = sid * ROWS_PER_TILE
    for t in range(ROWS_PER_TILE // CHUNK):
        pltpu.sync_copy(gbuf, acc.at[pl.ds(base + t * CHUNK, CHUNK)])
    plsc.subcore_barrier()

    @pl.loop(0, CHUNK)
    def _(i):
        for j in range(D // 16):
            gbuf[i, pl.ds(j * 16, 16)] = jnp.full((16,), 1.0, jnp.float32)

    pltpu.sync_copy(cols_hbm.at[wid], colbuf)

    @pl.loop(0, NCHUNK)
    def _(j):
        pltpu.sync_copy(gbuf, acc.at[colbuf.at[j]], add=True)

    plsc.subcore_barrier()
    for t in range(ROWS_PER_TILE // CHUNK):
        r0 = base + t * CHUNK
        pltpu.sync_copy(acc.at[pl.ds(r0, CHUNK)], gbuf)
        pltpu.sync_copy(gbuf, out_hbm.at[cid, pl.ds(r0, CHUNK)])


@functools.partial(
    pl.kernel,
    out_type=jax.ShapeDtypeStruct((NC, N_PAD, D), jnp.float32),
    mesh=_mesh,
    scratch_types=[
        pltpu.VMEM((NCHUNK, CHUNK), jnp.int32),
        pltpu.VMEM((NCHUNK, CHUNK), jnp.int32),
        pltpu.VMEM((CHUNK, D), jnp.float32),
        pltpu.VMEM_SHARED((N_PAD, D), jnp.float32),
        pltpu.SemaphoreType.DMA,
    ],
)
def _step_kernel(rows_hbm, cols_hbm, v_hbm, out_hbm,
                 rowbuf, colbuf, gbuf, acc, sem):
    cid = lax.axis_index("c")
    sid = lax.axis_index("s")
    wid = cid * NS + sid

    @pl.loop(0, CHUNK)
    def _(i):
        for j in range(D // 16):
            gbuf[i, pl.ds(j * 16, 16)] = jnp.zeros((16,), jnp.float32)

    base = sid * ROWS_PER_TILE
    for t in range(ROWS_PER_TILE // CHUNK):
        pltpu.sync_copy(gbuf, acc.at[pl.ds(base + t * CHUNK, CHUNK)])
    plsc.subcore_barrier()

    pltpu.sync_copy(rows_hbm.at[wid], rowbuf)
    pltpu.sync_copy(cols_hbm.at[wid], colbuf)

    @pl.loop(0, NCHUNK)
    def _(j):
        pltpu.async_copy(v_hbm.at[rowbuf.at[j]], gbuf, sem).wait()
        pltpu.sync_copy(gbuf, acc.at[colbuf.at[j]], add=True)

    plsc.subcore_barrier()
    for t in range(ROWS_PER_TILE // CHUNK):
        r0 = base + t * CHUNK
        pltpu.sync_copy(acc.at[pl.ds(r0, CHUNK)], gbuf)
        pltpu.sync_copy(gbuf, out_hbm.at[cid, pl.ds(r0, CHUNK)])


BLK = 1024


def _mlp_body(x_ref, w1_ref, b1_ref, w2_ref, b2_ref, o_ref):
    x = x_ref[...]
    h = lax.dot_general(x, w1_ref[...], (((1,), (1,)), ((), ())),
                        preferred_element_type=jnp.float32,
                        precision=lax.Precision.HIGHEST)
    h = jnp.maximum(h + b1_ref[...], 0.0)
    o = lax.dot_general(h, w2_ref[...], (((1,), (1,)), ((), ())),
                        preferred_element_type=jnp.float32,
                        precision=lax.Precision.HIGHEST)
    o_ref[...] = o + b2_ref[...]


def _mlp(x, W1, b1, W2, b2):
    return pl.pallas_call(
        _mlp_body,
        grid=(N_PAD // BLK,),
        in_specs=[
            pl.BlockSpec((BLK, D), lambda i: (i, 0)),
            pl.BlockSpec((D, D), lambda i: (0, 0)),
            pl.BlockSpec((1, D), lambda i: (0, 0)),
            pl.BlockSpec((D, D), lambda i: (0, 0)),
            pl.BlockSpec((1, D), lambda i: (0, 0)),
        ],
        out_specs=pl.BlockSpec((BLK, D), lambda i: (i, 0)),
        out_shape=jax.ShapeDtypeStruct((N_PAD, D), jnp.float32),
    )(x, W1, b1.reshape(1, D), W2, b2.reshape(1, D))


def _prep_body(t0_ref, d0_ref, d1_ref, h_ref, dinvb_ref, v_ref, hid_ref):
    deg = d0_ref[:, 0:1] + d1_ref[:, 0:1] + 1.0
    dinv = jnp.where(deg > 0, lax.rsqrt(jnp.maximum(deg, 1e-12)), 0.0)
    db = jnp.broadcast_to(dinv, h_ref.shape)
    h = h_ref[...]
    dinvb_ref[...] = db
    v_ref[...] = db * h
    hid_ref[...] = t0_ref[0, 0] * h


def _prep(t0, deg0, deg1, h0):
    out = jax.ShapeDtypeStruct((N_PAD, D), jnp.float32)
    return pl.pallas_call(
        _prep_body,
        grid=(N_PAD // BLK,),
        in_specs=[
            pl.BlockSpec(memory_space=pltpu.SMEM),
            pl.BlockSpec((BLK, D), lambda i: (i, 0)),
            pl.BlockSpec((BLK, D), lambda i: (i, 0)),
            pl.BlockSpec((BLK, D), lambda i: (i, 0)),
        ],
        out_specs=[
            pl.BlockSpec((BLK, D), lambda i: (i, 0)),
            pl.BlockSpec((BLK, D), lambda i: (i, 0)),
            pl.BlockSpec((BLK, D), lambda i: (i, 0)),
        ],
        out_shape=[out, out, out],
    )(t0, deg0, deg1, h0)


def _update_body(tk_ref, s0_ref, s1_ref, v_ref, db_ref, hid_ref,
                 vout_ref, hidout_ref):
    db = db_ref[...]
    h = db * (s0_ref[...] + s1_ref[...] + v_ref[...])
    hidout_ref[...] = hid_ref[...] + tk_ref[0, 0] * h
    vout_ref[...] = db * h


def _update(tk, s0, s1, v, dinvb, hid):
    out = jax.ShapeDtypeStruct((N_PAD, D), jnp.float32)
    blk = pl.BlockSpec((BLK, D), lambda i: (i, 0))
    return pl.pallas_call(
        _update_body,
        grid=(N_PAD // BLK,),
        in_specs=[pl.BlockSpec(memory_space=pltpu.SMEM), blk, blk, blk, blk, blk],
        out_specs=[blk, blk],
        out_shape=[out, out],
    )(tk, s0, s1, v, dinvb, hid)


def kernel(x, edge_index, W1, b1, W2, b2, temp):
    rows3 = edge_index[0].reshape(NT, NCHUNK, CHUNK)
    cols3 = edge_index[1].reshape(NT, NCHUNK, CHUNK)

    xp = jnp.pad(x, ((0, N_PAD - N_NODES), (0, 0)))
    h0 = _mlp(xp, W1, b1, W2, b2)
    degs = _deg_kernel(cols3)
    dinvb, v, hid = _prep(temp[0].reshape(1, 1).astype(jnp.float32),
                          degs[0], degs[1], h0)
    for k in range(K_STEPS):
        s = _step_kernel(rows3, cols3, v)
        v, hid = _update(temp[k + 1].reshape(1, 1).astype(jnp.float32),
                         s[0], s[1], v, dinvb, hid)
    return hid[:N_NODES]


# speedup vs baseline: 18.7905x; 1.6670x over previous
import functools

import jax
import jax.numpy as jnp
from jax import lax
from jax.experimental import pallas as pl
from jax.experimental.pallas import tpu as pltpu
from jax.experimental.pallas import tpu_sc as plsc

N_NODES = 10000
N_PAD = 10240
N_EDGES = 320000
D = 128
K_STEPS = 10

NC = 2
NS = 16
NT = NC * NS
EDGES_PER_TILE = N_EDGES // NT
CHUNK = 128
EPT_PAD = 10240
NCHUNK = EPT_PAD // CHUNK
PHASES = 2
CPP = NCHUNK // PHASES
ROWS_PER_TILE = N_PAD // NS

_mesh = plsc.VectorSubcoreMesh(
    core_axis_name="c", subcore_axis_name="s", num_cores=NC, num_subcores=NS
)


@functools.partial(
    pl.kernel,
    out_type=jax.ShapeDtypeStruct((NC, N_PAD, D), jnp.float32),
    mesh=_mesh,
    scratch_types=[
        pltpu.VMEM((NCHUNK, CHUNK), jnp.int32),
        pltpu.VMEM((CHUNK, D), jnp.float32),
        pltpu.VMEM_SHARED((N_PAD, D), jnp.float32),
    ],
)
def _deg_kernel(cols_hbm, out_hbm, colbuf, gbuf, acc):
    cid = lax.axis_index("c")
    sid = lax.axis_index("s")
    wid = cid * NS + sid

    @pl.loop(0, CHUNK)
    def _(i):
        for j in range(D // 16):
            gbuf[i, pl.ds(j * 16, 16)] = jnp.zeros((16,), jnp.float32)

    base = sid * ROWS_PER_TILE
    for t in range(ROWS_PER_TILE // CHUNK):
        pltpu.sync_copy(gbuf, acc.at[pl.ds(base + t * CHUNK, CHUNK)])
    plsc.subcore_barrier()

    @pl.loop(0, CHUNK)
    def _(i):
        for j in range(D // 16):
            gbuf[i, pl.ds(j * 16, 16)] = jnp.full((16,), 1.0, jnp.float32)

    pltpu.sync_copy(cols_hbm.at[wid], colbuf)

    @pl.loop(0, NCHUNK)
    def _(j):
        pltpu.sync_copy(gbuf, acc.at[colbuf.at[j]], add=True)

    plsc.subcore_barrier()
    for t in range(ROWS_PER_TILE // CHUNK):
        r0 = base + t * CHUNK
        pltpu.sync_copy(acc.at[pl.ds(r0, CHUNK)], gbuf)
        pltpu.sync_copy(gbuf, out_hbm.at[cid, pl.ds(r0, CHUNK)])


NBUF = 2


@functools.partial(
    pl.kernel,
    out_type=jax.ShapeDtypeStruct((NC, N_PAD, D), jnp.float32),
    mesh=_mesh,
    scratch_types=[
        pltpu.VMEM((CPP, CHUNK), jnp.int32),
        pltpu.VMEM((CPP, CHUNK), jnp.int32),
        pltpu.VMEM((CHUNK, D), jnp.float32),
        pltpu.VMEM((CHUNK, D), jnp.float32),
        pltpu.VMEM_SHARED((N_PAD, D), jnp.float32),
        pltpu.SemaphoreType.DMA,
        pltpu.SemaphoreType.DMA,
    ],
)
def _step_kernel(rows_hbm, cols_hbm, v_hbm, out_hbm,
                 rowbuf, colbuf, zbuf, g1, acc, s0, s1):
    cid = lax.axis_index("c")
    sid = lax.axis_index("s")
    wid = cid * NS + sid
    gbufs = (zbuf, g1)
    sems = (s0, s1)

    @pl.loop(0, CHUNK)
    def _(i):
        for j in range(D // 16):
            zbuf[i, pl.ds(j * 16, 16)] = jnp.zeros((16,), jnp.float32)

    base = sid * ROWS_PER_TILE
    for t in range(ROWS_PER_TILE // CHUNK):
        pltpu.sync_copy(zbuf, acc.at[pl.ds(base + t * CHUNK, CHUNK)])
    plsc.subcore_barrier()

    for p in range(PHASES):
        pltpu.sync_copy(rows_hbm.at[wid, pl.ds(p * CPP, CPP)], rowbuf)
        pltpu.sync_copy(cols_hbm.at[wid, pl.ds(p * CPP, CPP)], colbuf)

        for b in range(NBUF):
            pltpu.async_copy(v_hbm.at[rowbuf.at[b]], gbufs[b], sems[b])

        @pl.loop(0, CPP - NBUF, step=NBUF)
        def _(j0):
            for b in range(NBUF):
                pltpu.make_async_copy(v_hbm.at[rowbuf.at[j0 + b]],
                                      gbufs[b], sems[b]).wait()
                pltpu.sync_copy(gbufs[b], acc.at[colbuf.at[j0 + b]], add=True)
                pltpu.async_copy(v_hbm.at[rowbuf.at[j0 + NBUF + b]],
                                 gbufs[b], sems[b])

        for b in range(NBUF):
            jj = CPP - NBUF + b
            pltpu.make_async_copy(v_hbm.at[rowbuf.at[jj]],
                                  gbufs[b], sems[b]).wait()
            pltpu.sync_copy(gbufs[b], acc.at[colbuf.at[jj]], add=True)

    plsc.subcore_barrier()
    for t in range(ROWS_PER_TILE // CHUNK):
        r0 = base + t * CHUNK
        pltpu.sync_copy(acc.at[pl.ds(r0, CHUNK)], zbuf)
        pltpu.sync_copy(zbuf, out_hbm.at[cid, pl.ds(r0, CHUNK)])


BLK = 1024


def _mlp_body(x_ref, w1_ref, b1_ref, w2_ref, b2_ref, o_ref):
    x = x_ref[...]
    h = lax.dot_general(x, w1_ref[...], (((1,), (1,)), ((), ())),
                        preferred_element_type=jnp.float32,
                        precision=lax.Precision.HIGHEST)
    h = jnp.maximum(h + b1_ref[...], 0.0)
    o = lax.dot_general(h, w2_ref[...], (((1,), (1,)), ((), ())),
                        preferred_element_type=jnp.float32,
                        precision=lax.Precision.HIGHEST)
    o_ref[...] = o + b2_ref[...]


def _mlp(x, W1, b1, W2, b2):
    return pl.pallas_call(
        _mlp_body,
        grid=(N_PAD // BLK,),
        in_specs=[
            pl.BlockSpec((BLK, D), lambda i: (i, 0)),
            pl.BlockSpec((D, D), lambda i: (0, 0)),
            pl.BlockSpec((1, D), lambda i: (0, 0)),
            pl.BlockSpec((D, D), lambda i: (0, 0)),
            pl.BlockSpec((1, D), lambda i: (0, 0)),
        ],
        out_specs=pl.BlockSpec((BLK, D), lambda i: (i, 0)),
        out_shape=jax.ShapeDtypeStruct((N_PAD, D), jnp.float32),
    )(x, W1, b1.reshape(1, D), W2, b2.reshape(1, D))


def _prep_body(t0_ref, d0_ref, d1_ref, h_ref, dinvb_ref, v_ref, hid_ref):
    deg = d0_ref[:, 0:1] + d1_ref[:, 0:1] + 1.0
    dinv = jnp.where(deg > 0, lax.rsqrt(jnp.maximum(deg, 1e-12)), 0.0)
    db = jnp.broadcast_to(dinv, h_ref.shape)
    h = h_ref[...]
    dinvb_ref[...] = db
    v_ref[...] = db * h
    hid_ref[...] = t0_ref[0, 0] * h


def _prep(t0, deg0, deg1, h0):
    out = jax.ShapeDtypeStruct((N_PAD, D), jnp.float32)
    return pl.pallas_call(
        _prep_body,
        grid=(N_PAD // BLK,),
        in_specs=[
            pl.BlockSpec(memory_space=pltpu.SMEM),
            pl.BlockSpec((BLK, D), lambda i: (i, 0)),
            pl.BlockSpec((BLK, D), lambda i: (i, 0)),
            pl.BlockSpec((BLK, D), lambda i: (i, 0)),
        ],
        out_specs=[
            pl.BlockSpec((BLK, D), lambda i: (i, 0)),
            pl.BlockSpec((BLK, D), lambda i: (i, 0)),
            pl.BlockSpec((BLK, D), lambda i: (i, 0)),
        ],
        out_shape=[out, out, out],
    )(t0, deg0, deg1, h0)


def _update_body(tk_ref, s0_ref, s1_ref, v_ref, db_ref, hid_ref,
                 vout_ref, hidout_ref):
    db = db_ref[...]
    h = db * (s0_ref[...] + s1_ref[...] + v_ref[...])
    hidout_ref[...] = hid_ref[...] + tk_ref[0, 0] * h
    vout_ref[...] = db * h


def _update(tk, s0, s1, v, dinvb, hid):
    out = jax.ShapeDtypeStruct((N_PAD, D), jnp.float32)
    blk = pl.BlockSpec((BLK, D), lambda i: (i, 0))
    return pl.pallas_call(
        _update_body,
        grid=(N_PAD // BLK,),
        in_specs=[pl.BlockSpec(memory_space=pltpu.SMEM), blk, blk, blk, blk, blk],
        out_specs=[blk, blk],
        out_shape=[out, out],
    )(tk, s0, s1, v, dinvb, hid)


def kernel(x, edge_index, W1, b1, W2, b2, temp):
    dummy = (N_NODES + jnp.arange(EPT_PAD - EDGES_PER_TILE, dtype=jnp.int32)
             % (N_PAD - N_NODES))
    dummy = jnp.broadcast_to(dummy, (NT, EPT_PAD - EDGES_PER_TILE))

    def _tile(idx):
        per_tile = idx.reshape(NT, EDGES_PER_TILE)
        return jnp.concatenate([per_tile, dummy], axis=1).reshape(
            NT, NCHUNK, CHUNK)

    rows3 = _tile(edge_index[0])
    cols3 = _tile(edge_index[1])

    xp = jnp.pad(x, ((0, N_PAD - N_NODES), (0, 0)))
    h0 = _mlp(xp, W1, b1, W2, b2)
    degs = _deg_kernel(cols3)
    dinvb, v, hid = _prep(temp[0].reshape(1, 1).astype(jnp.float32),
                          degs[0], degs[1], h0)
    for k in range(K_STEPS):
        s = _step_kernel(rows3, cols3, v)
        v, hid = _update(temp[k + 1].reshape(1, 1).astype(jnp.float32),
                         s[0], s[1], v, dinvb, hid)
    return hid[:N_NODES]


# speedup vs baseline: 18.9410x; 1.0080x over previous
import functools

import jax
import jax.numpy as jnp
from jax import lax
from jax.experimental import pallas as pl
from jax.experimental.pallas import tpu as pltpu
from jax.experimental.pallas import tpu_sc as plsc

N_NODES = 10000
N_PAD = 10240
N_EDGES = 320000
D = 128
K_STEPS = 10

NC = 2
NS = 16
NT = NC * NS
EDGES_PER_TILE = N_EDGES // NT
CHUNK = 128
EPT_PAD = 10240
NCHUNK = EPT_PAD // CHUNK
PHASES = 2
CPP = NCHUNK // PHASES
ROWS_PER_TILE = N_PAD // NS

_mesh = plsc.VectorSubcoreMesh(
    core_axis_name="c", subcore_axis_name="s", num_cores=NC, num_subcores=NS
)


@functools.partial(
    pl.kernel,
    out_type=jax.ShapeDtypeStruct((NC, N_PAD, D), jnp.float32),
    mesh=_mesh,
    scratch_types=[
        pltpu.VMEM((NCHUNK, CHUNK), jnp.int32),
        pltpu.VMEM((CHUNK, D), jnp.float32),
        pltpu.VMEM_SHARED((N_PAD, D), jnp.float32),
    ],
)
def _deg_kernel(cols_hbm, out_hbm, colbuf, gbuf, acc):
    cid = lax.axis_index("c")
    sid = lax.axis_index("s")
    wid = cid * NS + sid

    @pl.loop(0, CHUNK)
    def _(i):
        for j in range(D // 16):
            gbuf[i, pl.ds(j * 16, 16)] = jnp.zeros((16,), jnp.float32)

    base = sid * ROWS_PER_TILE
    for t in range(ROWS_PER_TILE // CHUNK):
        pltpu.sync_copy(gbuf, acc.at[pl.ds(base + t * CHUNK, CHUNK)])
    plsc.subcore_barrier()

    @pl.loop(0, CHUNK)
    def _(i):
        for j in range(D // 16):
            gbuf[i, pl.ds(j * 16, 16)] = jnp.full((16,), 1.0, jnp.float32)

    pltpu.sync_copy(cols_hbm.at[wid], colbuf)

    @pl.loop(0, NCHUNK)
    def _(j):
        pltpu.sync_copy(gbuf, acc.at[colbuf.at[j]], add=True)

    plsc.subcore_barrier()
    pltpu.sync_copy(acc.at[pl.ds(base, ROWS_PER_TILE)],
                    out_hbm.at[cid, pl.ds(base, ROWS_PER_TILE)])


NBUF = 2


@functools.partial(
    pl.kernel,
    out_type=jax.ShapeDtypeStruct((NC, N_PAD, D), jnp.float32),
    mesh=_mesh,
    scratch_types=[
        pltpu.VMEM((CPP, CHUNK), jnp.int32),
        pltpu.VMEM((CPP, CHUNK), jnp.int32),
        pltpu.VMEM((CHUNK, D), jnp.float32),
        pltpu.VMEM((CHUNK, D), jnp.float32),
        pltpu.VMEM_SHARED((N_PAD, D), jnp.float32),
        pltpu.SemaphoreType.DMA,
        pltpu.SemaphoreType.DMA,
    ],
)
def _step_kernel(rows_hbm, cols_hbm, v_hbm, out_hbm,
                 rowbuf, colbuf, zbuf, g1, acc, s0, s1):
    cid = lax.axis_index("c")
    sid = lax.axis_index("s")
    wid = cid * NS + sid
    gbufs = (zbuf, g1)
    sems = (s0, s1)

    @pl.loop(0, CHUNK)
    def _(i):
        for j in range(D // 16):
            zbuf[i, pl.ds(j * 16, 16)] = jnp.zeros((16,), jnp.float32)

    base = sid * ROWS_PER_TILE
    for t in range(ROWS_PER_TILE // CHUNK):
        pltpu.sync_copy(zbuf, acc.at[pl.ds(base + t * CHUNK, CHUNK)])
    plsc.subcore_barrier()

    for p in range(PHASES):
        pltpu.sync_copy(rows_hbm.at[wid, pl.ds(p * CPP, CPP)], rowbuf)
        pltpu.sync_copy(cols_hbm.at[wid, pl.ds(p * CPP, CPP)], colbuf)

        for b in range(NBUF):
            pltpu.async_copy(v_hbm.at[rowbuf.at[b]], gbufs[b], sems[b])

        @pl.loop(0, CPP - NBUF, step=NBUF)
        def _(j0):
            for b in range(NBUF):
                pltpu.make_async_copy(v_hbm.at[rowbuf.at[j0 + b]],
                                      gbufs[b], sems[b]).wait()
                pltpu.sync_copy(gbufs[b], acc.at[colbuf.at[j0 + b]], add=True)
                pltpu.async_copy(v_hbm.at[rowbuf.at[j0 + NBUF + b]],
                                 gbufs[b], sems[b])

        for b in range(NBUF):
            jj = CPP - NBUF + b
            pltpu.make_async_copy(v_hbm.at[rowbuf.at[jj]],
                                  gbufs[b], sems[b]).wait()
            pltpu.sync_copy(gbufs[b], acc.at[colbuf.at[jj]], add=True)

    plsc.subcore_barrier()
    pltpu.sync_copy(acc.at[pl.ds(base, ROWS_PER_TILE)],
                    out_hbm.at[cid, pl.ds(base, ROWS_PER_TILE)])


BLK = 1024


def _mlp_body(x_ref, w1_ref, b1_ref, w2_ref, b2_ref, o_ref):
    x = x_ref[...]
    h = lax.dot_general(x, w1_ref[...], (((1,), (1,)), ((), ())),
                        preferred_element_type=jnp.float32,
                        precision=lax.Precision.HIGHEST)
    h = jnp.maximum(h + b1_ref[...], 0.0)
    o = lax.dot_general(h, w2_ref[...], (((1,), (1,)), ((), ())),
                        preferred_element_type=jnp.float32,
                        precision=lax.Precision.HIGHEST)
    o_ref[...] = o + b2_ref[...]


def _mlp(x, W1, b1, W2, b2):
    return pl.pallas_call(
        _mlp_body,
        grid=(N_PAD // BLK,),
        in_specs=[
            pl.BlockSpec((BLK, D), lambda i: (i, 0)),
            pl.BlockSpec((D, D), lambda i: (0, 0)),
            pl.BlockSpec((1, D), lambda i: (0, 0)),
            pl.BlockSpec((D, D), lambda i: (0, 0)),
            pl.BlockSpec((1, D), lambda i: (0, 0)),
        ],
        out_specs=pl.BlockSpec((BLK, D), lambda i: (i, 0)),
        out_shape=jax.ShapeDtypeStruct((N_PAD, D), jnp.float32),
    )(x, W1, b1.reshape(1, D), W2, b2.reshape(1, D))


def _prep_body(d0_ref, d1_ref, h_ref, d2_ref, v_ref):
    deg = d0_ref[:, 0:1] + d1_ref[:, 0:1] + 1.0
    d2_ref[...] = jnp.broadcast_to(1.0 / deg, h_ref.shape)
    v_ref[...] = jnp.broadcast_to(lax.rsqrt(deg), h_ref.shape) * h_ref[...]


def _prep(deg0, deg1, h0):
    out = jax.ShapeDtypeStruct((N_PAD, D), jnp.float32)
    blk = pl.BlockSpec((BLK, D), lambda i: (i, 0))
    return pl.pallas_call(
        _prep_body,
        grid=(N_PAD // BLK,),
        in_specs=[blk, blk, blk],
        out_specs=[blk, blk],
        out_shape=[out, out],
    )(deg0, deg1, h0)


def _vnext_body(s0_ref, s1_ref, v_ref, d2_ref, vout_ref):
    vout_ref[...] = d2_ref[...] * (s0_ref[...] + s1_ref[...] + v_ref[...])


def _vnext(s0, s1, v, d2b):
    out = jax.ShapeDtypeStruct((N_PAD, D), jnp.float32)
    blk = pl.BlockSpec((BLK, D), lambda i: (i, 0))
    return pl.pallas_call(
        _vnext_body,
        grid=(N_PAD // BLK,),
        in_specs=[blk, blk, blk, blk],
        out_specs=blk,
        out_shape=out,
    )(s0, s1, v, d2b)


def _fin_body(t_ref, d0_ref, d1_ref, *refs):
    vrefs, o_ref = refs[:-1], refs[-1]
    acc = t_ref[0, 0] * vrefs[0][...]
    for k in range(1, K_STEPS + 1):
        acc = acc + t_ref[0, k] * vrefs[k][...]
    deg = d0_ref[:, 0:1] + d1_ref[:, 0:1] + 1.0
    o_ref[...] = jnp.broadcast_to(jnp.sqrt(deg), acc.shape) * acc


def _fin(temp, deg0, deg1, vs):
    out = jax.ShapeDtypeStruct((N_PAD, D), jnp.float32)
    blk = pl.BlockSpec((BLK, D), lambda i: (i, 0))
    return pl.pallas_call(
        _fin_body,
        grid=(N_PAD // BLK,),
        in_specs=[pl.BlockSpec(memory_space=pltpu.SMEM)]
        + [blk, blk] + [blk] * len(vs),
        out_specs=blk,
        out_shape=out,
    )(temp, deg0, deg1, *vs)


def kernel(x, edge_index, W1, b1, W2, b2, temp):
    dummy = (N_NODES + jnp.arange(EPT_PAD - EDGES_PER_TILE, dtype=jnp.int32)
             % (N_PAD - N_NODES))
    dummy = jnp.broadcast_to(dummy, (NT, EPT_PAD - EDGES_PER_TILE))

    def _tile(idx):
        per_tile = idx.reshape(NT, EDGES_PER_TILE)
        return jnp.concatenate([per_tile, dummy], axis=1).reshape(
            NT, NCHUNK, CHUNK)

    rows3 = _tile(edge_index[0])
    cols3 = _tile(edge_index[1])

    xp = jnp.pad(x, ((0, N_PAD - N_NODES), (0, 0)))
    h0 = _mlp(xp, W1, b1, W2, b2)
    degs = _deg_kernel(cols3)
    d2b, v = _prep(degs[0], degs[1], h0)
    vs = [v]
    for k in range(K_STEPS):
        s = _step_kernel(rows3, cols3, v)
        v = _vnext(s[0], s[1], v, d2b)
        vs.append(v)
    hid = _fin(temp.reshape(1, K_STEPS + 1), degs[0], degs[1], vs)
    return hid[:N_NODES]


# speedup vs baseline: 19.4489x; 1.0268x over previous
import functools

import jax
import jax.numpy as jnp
from jax import lax
from jax.experimental import pallas as pl
from jax.experimental.pallas import tpu as pltpu
from jax.experimental.pallas import tpu_sc as plsc

N_NODES = 10000
N_PAD = 10240
N_EDGES = 320000
D = 128
K_STEPS = 10

NC = 2
NS = 16
NT = NC * NS
EDGES_PER_TILE = N_EDGES // NT
CHUNK = 128
EPT_PAD = 10240
NCHUNK = EPT_PAD // CHUNK
PHASES = 2
CPP = NCHUNK // PHASES
ROWS_PER_TILE = N_PAD // NS

_mesh = plsc.VectorSubcoreMesh(
    core_axis_name="c", subcore_axis_name="s", num_cores=NC, num_subcores=NS
)


DEG_W = 16


@functools.partial(
    pl.kernel,
    out_type=jax.ShapeDtypeStruct((NC, N_PAD, DEG_W), jnp.float32),
    mesh=_mesh,
    scratch_types=[
        pltpu.VMEM((NCHUNK, CHUNK), jnp.int32),
        pltpu.VMEM((CHUNK, DEG_W), jnp.float32),
        pltpu.VMEM_SHARED((N_PAD, DEG_W), jnp.float32),
    ],
)
def _deg_kernel(cols_hbm, out_hbm, colbuf, gbuf, acc):
    cid = lax.axis_index("c")
    sid = lax.axis_index("s")
    wid = cid * NS + sid

    @pl.loop(0, CHUNK)
    def _(i):
        gbuf[i, :] = jnp.zeros((DEG_W,), jnp.float32)

    base = sid * ROWS_PER_TILE
    for t in range(ROWS_PER_TILE // CHUNK):
        pltpu.sync_copy(gbuf, acc.at[pl.ds(base + t * CHUNK, CHUNK)])
    plsc.subcore_barrier()

    @pl.loop(0, CHUNK)
    def _(i):
        gbuf[i, :] = jnp.full((DEG_W,), 1.0, jnp.float32)

    pltpu.sync_copy(cols_hbm.at[wid], colbuf)

    @pl.loop(0, NCHUNK)
    def _(j):
        pltpu.sync_copy(gbuf, acc.at[colbuf.at[j]], add=True)

    plsc.subcore_barrier()
    pltpu.sync_copy(acc.at[pl.ds(base, ROWS_PER_TILE)],
                    out_hbm.at[cid, pl.ds(base, ROWS_PER_TILE)])


NBUF = 2


@functools.partial(
    pl.kernel,
    out_type=jax.ShapeDtypeStruct((NC, N_PAD, D), jnp.float32),
    mesh=_mesh,
    scratch_types=[
        pltpu.VMEM((CPP, CHUNK), jnp.int32),
        pltpu.VMEM((CPP, CHUNK), jnp.int32),
        pltpu.VMEM((CHUNK, D), jnp.float32),
        pltpu.VMEM((CHUNK, D), jnp.float32),
        pltpu.VMEM_SHARED((N_PAD, D), jnp.float32),
        pltpu.SemaphoreType.DMA,
        pltpu.SemaphoreType.DMA,
    ],
)
def _step_kernel(rows_hbm, cols_hbm, v_hbm, out_hbm,
                 rowbuf, colbuf, zbuf, g1, acc, s0, s1):
    cid = lax.axis_index("c")
    sid = lax.axis_index("s")
    wid = cid * NS + sid
    gbufs = (zbuf, g1)
    sems = (s0, s1)

    @pl.loop(0, CHUNK)
    def _(i):
        for j in range(D // 16):
            zbuf[i, pl.ds(j * 16, 16)] = jnp.zeros((16,), jnp.float32)

    base = sid * ROWS_PER_TILE
    for t in range(ROWS_PER_TILE // CHUNK):
        pltpu.sync_copy(zbuf, acc.at[pl.ds(base + t * CHUNK, CHUNK)])
    plsc.subcore_barrier()

    for p in range(PHASES):
        pltpu.sync_copy(rows_hbm.at[wid, pl.ds(p * CPP, CPP)], rowbuf)
        pltpu.sync_copy(cols_hbm.at[wid, pl.ds(p * CPP, CPP)], colbuf)

        for b in range(NBUF):
            pltpu.async_copy(v_hbm.at[rowbuf.at[b]], gbufs[b], sems[b])

        @pl.loop(0, CPP - NBUF, step=NBUF)
        def _(j0):
            for b in range(NBUF):
                pltpu.make_async_copy(v_hbm.at[rowbuf.at[j0 + b]],
                                      gbufs[b], sems[b]).wait()
                pltpu.sync_copy(gbufs[b], acc.at[colbuf.at[j0 + b]], add=True)
                pltpu.async_copy(v_hbm.at[rowbuf.at[j0 + NBUF + b]],
                                 gbufs[b], sems[b])

        for b in range(NBUF):
            jj = CPP - NBUF + b
            pltpu.make_async_copy(v_hbm.at[rowbuf.at[jj]],
                                  gbufs[b], sems[b]).wait()
            pltpu.sync_copy(gbufs[b], acc.at[colbuf.at[jj]], add=True)

    plsc.subcore_barrier()
    pltpu.sync_copy(acc.at[pl.ds(base, ROWS_PER_TILE)],
                    out_hbm.at[cid, pl.ds(base, ROWS_PER_TILE)])


BLK = 1024


def _mlp_body(x_ref, w1_ref, b1_ref, w2_ref, b2_ref, o_ref):
    x = x_ref[...]
    h = lax.dot_general(x, w1_ref[...], (((1,), (1,)), ((), ())),
                        preferred_element_type=jnp.float32,
                        precision=lax.Precision.HIGHEST)
    h = jnp.maximum(h + b1_ref[...], 0.0)
    o = lax.dot_general(h, w2_ref[...], (((1,), (1,)), ((), ())),
                        preferred_element_type=jnp.float32,
                        precision=lax.Precision.HIGHEST)
    o_ref[...] = o + b2_ref[...]


def _mlp(x, W1, b1, W2, b2):
    return pl.pallas_call(
        _mlp_body,
        grid=(N_PAD // BLK,),
        in_specs=[
            pl.BlockSpec((BLK, D), lambda i: (i, 0)),
            pl.BlockSpec((D, D), lambda i: (0, 0)),
            pl.BlockSpec((1, D), lambda i: (0, 0)),
            pl.BlockSpec((D, D), lambda i: (0, 0)),
            pl.BlockSpec((1, D), lambda i: (0, 0)),
        ],
        out_specs=pl.BlockSpec((BLK, D), lambda i: (i, 0)),
        out_shape=jax.ShapeDtypeStruct((N_PAD, D), jnp.float32),
    )(x, W1, b1.reshape(1, D), W2, b2.reshape(1, D))


def _prep_body(d0_ref, d1_ref, h_ref, d2_ref, v_ref):
    deg = d0_ref[:, 0:1] + d1_ref[:, 0:1] + 1.0
    d2_ref[...] = jnp.broadcast_to(1.0 / deg, h_ref.shape)
    v_ref[...] = jnp.broadcast_to(lax.rsqrt(deg), h_ref.shape) * h_ref[...]


def _prep(deg0, deg1, h0):
    out = jax.ShapeDtypeStruct((N_PAD, D), jnp.float32)
    blk = pl.BlockSpec((BLK, D), lambda i: (i, 0))
    dblk = pl.BlockSpec((BLK, DEG_W), lambda i: (i, 0))
    return pl.pallas_call(
        _prep_body,
        grid=(N_PAD // BLK,),
        in_specs=[dblk, dblk, blk],
        out_specs=[blk, blk],
        out_shape=[out, out],
    )(deg0, deg1, h0)


def _vnext_body(s0_ref, s1_ref, v_ref, d2_ref, vout_ref):
    vout_ref[...] = d2_ref[...] * (s0_ref[...] + s1_ref[...] + v_ref[...])


def _vnext(s0, s1, v, d2b):
    out = jax.ShapeDtypeStruct((N_PAD, D), jnp.float32)
    blk = pl.BlockSpec((BLK, D), lambda i: (i, 0))
    return pl.pallas_call(
        _vnext_body,
        grid=(N_PAD // BLK,),
        in_specs=[blk, blk, blk, blk],
        out_specs=blk,
        out_shape=out,
    )(s0, s1, v, d2b)


def _fin_body(t_ref, d0_ref, d1_ref, *refs):
    vrefs, o_ref = refs[:-1], refs[-1]
    acc = t_ref[0, 0] * vrefs[0][...]
    for k in range(1, K_STEPS + 1):
        acc = acc + t_ref[0, k] * vrefs[k][...]
    deg = d0_ref[:, 0:1] + d1_ref[:, 0:1] + 1.0
    o_ref[...] = jnp.broadcast_to(jnp.sqrt(deg), acc.shape) * acc


def _fin(temp, deg0, deg1, vs):
    out = jax.ShapeDtypeStruct((N_PAD, D), jnp.float32)
    blk = pl.BlockSpec((BLK, D), lambda i: (i, 0))
    dblk = pl.BlockSpec((BLK, DEG_W), lambda i: (i, 0))
    return pl.pallas_call(
        _fin_body,
        grid=(N_PAD // BLK,),
        in_specs=[pl.BlockSpec(memory_space=pltpu.SMEM)]
        + [dblk, dblk] + [blk] * len(vs),
        out_specs=blk,
        out_shape=out,
    )(temp, deg0, deg1, *vs)


def kernel(x, edge_index, W1, b1, W2, b2, temp):
    dummy = (N_NODES + jnp.arange(EPT_PAD - EDGES_PER_TILE, dtype=jnp.int32)
             % (N_PAD - N_NODES))
    dummy = jnp.broadcast_to(dummy, (NT, EPT_PAD - EDGES_PER_TILE))

    def _tile(idx):
        per_tile = idx.reshape(NT, EDGES_PER_TILE)
        return jnp.concatenate([per_tile, dummy], axis=1).reshape(
            NT, NCHUNK, CHUNK)

    rows3 = _tile(edge_index[0])
    cols3 = _tile(edge_index[1])

    xp = jnp.pad(x, ((0, N_PAD - N_NODES), (0, 0)))
    h0 = _mlp(xp, W1, b1, W2, b2)
    degs = _deg_kernel(cols3)
    d2b, v = _prep(degs[0], degs[1], h0)
    vs = [v]
    for k in range(K_STEPS):
        s = _step_kernel(rows3, cols3, v)
        v = _vnext(s[0], s[1], v, d2b)
        vs.append(v)
    hid = _fin(temp.reshape(1, K_STEPS + 1), degs[0], degs[1], vs)
    return hid[:N_NODES]
